# baseline (speedup 1.0000x reference)
import functools

import jax
import jax.numpy as jnp
import numpy as np
from jax import lax
from jax.experimental import pallas as pl
from jax.experimental.pallas import tpu as pltpu
from jax.experimental.pallas import tpu_sc as plsc

_VOCAB = 100000
_D = 768
_MAX_LEN = 2048
_B = 4
_L = 2048

_NC = 2
_NS = 16
_NW = _NC * _NS

_ROWS = _B * _L
_POS_W = _L // _NW
_LANES = 16
_JSTEPS = _D // _LANES


def _pos_encoding() -> np.ndarray:
    pos = np.arange(_MAX_LEN, dtype=np.float64)[:, None]
    idx = np.arange(0, _D, 2, dtype=np.float64)[None, :]
    angle = pos / np.power(10000.0, idx / float(_D))
    pe = np.zeros((_MAX_LEN, _D), dtype=np.float64)
    pe[:, 0::2] = np.sin(angle)
    pe[:, 1::2] = np.cos(angle)
    return pe.astype(np.float32)


_PE = _pos_encoding()


def _emb_body(x_hbm, table_hbm, pe_hbm, out_hbm, idx_v, rows_v, pe_v, sem):
    wid = lax.axis_index("s") * _NC + lax.axis_index("c")
    pos0 = wid * _POS_W
    pltpu.sync_copy(pe_hbm.at[pl.ds(pos0, _POS_W)], pe_v)
    for b in range(_B):
        base = b * _L + pos0
        pltpu.sync_copy(x_hbm.at[pl.ds(base, _POS_W)], idx_v)
        pltpu.async_copy(table_hbm.at[idx_v], rows_v, sem).wait()

        @pl.loop(0, _POS_W)
        def _row(r):
            for j in range(_JSTEPS):
                sl = pl.ds(j * _LANES, _LANES)
                rows_v[r, sl] = rows_v[r, sl] + pe_v[r, sl]

        pltpu.sync_copy(rows_v, out_hbm.at[pl.ds(base, _POS_W)])


@jax.jit
def _sc_embed(x_flat, table, pe):
    mesh = plsc.VectorSubcoreMesh(
        core_axis_name="c", subcore_axis_name="s",
        num_cores=_NC, num_subcores=_NS,
    )
    fn = pl.kernel(
        _emb_body,
        out_type=jax.ShapeDtypeStruct((_ROWS, _D), jnp.float32),
        mesh=mesh,
        scratch_types=[
            pltpu.VMEM((_POS_W,), jnp.int32),
            pltpu.VMEM((_POS_W, _D), jnp.float32),
            pltpu.VMEM((_POS_W, _D), jnp.float32),
            pltpu.SemaphoreType.DMA,
        ],
    )
    return fn(x_flat, table, pe)


def kernel(x, table):
    pe = jnp.asarray(_PE)
    out = _sc_embed(x.reshape(_ROWS), table, pe)
    return out.reshape(_B, _L, _D)
